# baseline (speedup 1.0000x reference)
import functools

import jax
import jax.numpy as jnp
from jax.experimental import pallas as pl
from jax.experimental.pallas import tpu as pltpu


_DN_T = (((1,), (1,)), ((), ()))


def _se_kernel(x_ref, w1_ref, b1_ref, w2_ref, b2_ref, o_ref, *, inv_hw, imgs):
    for i in range(imgs):
        xb = x_ref[i]
        s1 = jnp.sum(xb.astype(jnp.float32), axis=0)
        s = jnp.sum(s1, axis=0, keepdims=True)
        p = s * inv_hw
        y1 = jax.lax.dot_general(p, w1_ref[...], _DN_T,
                                 preferred_element_type=jnp.float32)
        y1 = jnp.maximum(y1 + b1_ref[...], 0.0)
        y2 = jnp.dot(y1, w2_ref[...], preferred_element_type=jnp.float32)
        g = jnp.tanh(y2 + b2_ref[...])
        o_ref[i] = xb * g.astype(o_ref.dtype)


def kernel(x, w1, b1, w2, b2):
    B, C, H, W = x.shape
    R = w1.shape[0]

    xt = jnp.transpose(x, (0, 2, 3, 1))

    w2t = jnp.transpose(w2)
    b1r = b1.astype(jnp.float32).reshape(1, R)
    b2r = b2.astype(jnp.float32).reshape(1, C)

    imgs = 4
    while B % imgs:
        imgs //= 2

    out_t = pl.pallas_call(
        functools.partial(_se_kernel, inv_hw=1.0 / float(H * W), imgs=imgs),
        out_shape=jax.ShapeDtypeStruct((B, H, W, C), x.dtype),
        grid=(B // imgs,),
        in_specs=[
            pl.BlockSpec((imgs, H, W, C), lambda b: (b, 0, 0, 0)),
            pl.BlockSpec((R, C), lambda b: (0, 0)),
            pl.BlockSpec((1, R), lambda b: (0, 0)),
            pl.BlockSpec((R, C), lambda b: (0, 0)),
            pl.BlockSpec((1, C), lambda b: (0, 0)),
        ],
        out_specs=pl.BlockSpec((imgs, H, W, C), lambda b: (b, 0, 0, 0)),
        compiler_params=pltpu.CompilerParams(
            dimension_semantics=("parallel",)),
        cost_estimate=pl.CostEstimate(
            flops=2 * B * C * H * W + 4 * B * C * R,
            transcendentals=B * C,
            bytes_accessed=2 * B * C * H * W * x.dtype.itemsize),
    )(xt, w1, b1r, w2t, b2r)

    return jnp.transpose(out_t, (0, 3, 1, 2))
